# Initial kernel scaffold; baseline (speedup 1.0000x reference)
import jax
import jax.numpy as jnp
from jax.experimental import pallas as pl


def kernel(x, seg, token_emb, pos_emb, gamma, beta):
    raise NotImplementedError("write your pallas kernel here")


# speedup vs baseline: 3.9960x; 3.9960x over previous
import jax
import jax.numpy as jnp
from jax import lax
from jax.experimental import pallas as pl
from jax.experimental.pallas import tpu as pltpu
from jax.experimental.pallas import tpu_sc as plsc

NC = 2
NS = 16
NW = NC * NS
LANES = 16


def _rsqrt_newton(v):
    bits = lax.bitcast_convert_type(v, jnp.int32)
    seed = jnp.int32(0x5F3759DF) - lax.shift_right_logical(bits, 1)
    y = lax.bitcast_convert_type(seed, jnp.float32)
    for _ in range(3):
        y = y * (jnp.float32(1.5) - jnp.float32(0.5) * v * y * y)
    return y


def kernel(x, seg, token_emb, pos_emb, gamma, beta):
    B, L = x.shape
    V, D = token_emb.shape
    NV = D // LANES
    assert L == 200 and D == 128 and (B % NW) == 0
    SEQ_PER_W = B // NW
    HALF = L // 2

    x2 = x.astype(jnp.int32).reshape(B * 2, HALF)
    seg1 = seg.astype(jnp.int32).reshape(B * L)

    mesh = plsc.VectorSubcoreMesh(
        core_axis_name="c", subcore_axis_name="s",
        num_cores=NC, num_subcores=NS)

    @pl.kernel(
        out_type=jax.ShapeDtypeStruct((B, L, D), jnp.float32),
        mesh=mesh,
        scratch_types=[
            pltpu.VMEM((2, HALF), jnp.int32),
            pltpu.VMEM((L + LANES,), jnp.int32),
            pltpu.VMEM((L, D), jnp.float32),
            pltpu.VMEM((L, D), jnp.float32),
            pltpu.VMEM((D,), jnp.float32),
            pltpu.VMEM((D,), jnp.float32),
            pltpu.SemaphoreType.DMA,
        ],
        compiler_params=pltpu.CompilerParams(needs_layout_passes=False),
    )
    def body(x_hbm, seg_hbm, tok_hbm, pos_hbm, gamma_hbm, beta_hbm, out_hbm,
             idx_v, seg_v, buf, pos_v, gamma_v, beta_v, sem):
        wid = lax.axis_index("s") * NC + lax.axis_index("c")
        seq0 = wid * SEQ_PER_W

        pltpu.sync_copy(pos_hbm.at[pl.ds(0, L)], pos_v)
        pltpu.sync_copy(gamma_hbm, gamma_v)
        pltpu.sync_copy(beta_hbm, beta_v)

        p0 = [pos_v[0, pl.ds(LANES * j, LANES)] for j in range(NV)]
        p1 = [pos_v[1, pl.ds(LANES * j, LANES)] for j in range(NV)]
        dseg = [p1[j] - p0[j] for j in range(NV)]
        gam = [gamma_v[pl.ds(LANES * j, LANES)] for j in range(NV)]
        bet = [beta_v[pl.ds(LANES * j, LANES)] for j in range(NV)]

        def fold(l, _):
            for j in range(NV):
                sl = pl.ds(LANES * j, LANES)
                pos_v[l, sl] = pos_v[l, sl] + p0[j]
            return 0
        lax.fori_loop(0, L, fold, 0)

        inv_d = jnp.float32(1.0 / D)
        eps = jnp.float32(1e-5)

        def row_body(i, _):
            s_f = seg_v[pl.ds(i, LANES)][0].astype(jnp.float32)
            sb = lax.broadcast_in_dim(s_f, (LANES,), ())
            h = []
            for j in range(NV):
                sl = pl.ds(LANES * j, LANES)
                h.append(buf[i, sl] + pos_v[i, sl] + sb * dseg[j])
            acc = (h[0] + h[1]) + (h[2] + h[3])
            acc = acc + ((h[4] + h[5]) + (h[6] + h[7]))
            accq = (h[0] * h[0] + h[1] * h[1]) + (h[2] * h[2] + h[3] * h[3])
            accq = accq + ((h[4] * h[4] + h[5] * h[5])
                           + (h[6] * h[6] + h[7] * h[7]))
            mu = jnp.sum(acc) * inv_d
            var = jnp.sum(accq) * inv_d - mu * mu
            r = _rsqrt_newton(var + eps)
            mub = lax.broadcast_in_dim(mu, (LANES,), ())
            rb = lax.broadcast_in_dim(r, (LANES,), ())
            for j in range(NV):
                sl = pl.ds(LANES * j, LANES)
                buf[i, sl] = (h[j] - mub) * rb * gam[j] + bet[j]
            return 0

        def seq_body(s, _):
            g = seq0 + s
            pltpu.sync_copy(x_hbm.at[pl.ds(2 * g, 2)], idx_v)
            pltpu.sync_copy(seg_hbm.at[pl.ds(L * g, L)], seg_v.at[pl.ds(0, L)])
            cp0 = pltpu.async_copy(
                tok_hbm.at[idx_v.at[0]], buf.at[pl.ds(0, HALF)], sem)
            cp1 = pltpu.async_copy(
                tok_hbm.at[idx_v.at[1]], buf.at[pl.ds(HALF, HALF)], sem)
            cp0.wait()
            cp1.wait()
            lax.fori_loop(0, L, row_body, 0)
            pltpu.sync_copy(buf, out_hbm.at[g])
            return 0

        lax.fori_loop(0, SEQ_PER_W, seq_body, 0)

    return body(x2, seg1, token_emb, pos_emb, gamma, beta)


# speedup vs baseline: 4.4947x; 1.1248x over previous
import jax
import jax.numpy as jnp
from jax import lax
from jax.experimental import pallas as pl
from jax.experimental.pallas import tpu as pltpu
from jax.experimental.pallas import tpu_sc as plsc

NC = 2
NS = 16
NW = NC * NS
LANES = 16
RING = 4


def _rsqrt_newton(v):
    bits = lax.bitcast_convert_type(v, jnp.int32)
    seed = jnp.int32(0x5F3759DF) - lax.shift_right_logical(bits, 1)
    y = lax.bitcast_convert_type(seed, jnp.float32)
    for _ in range(3):
        y = y * (jnp.float32(1.5) - jnp.float32(0.5) * v * y * y)
    return y


def kernel(x, seg, token_emb, pos_emb, gamma, beta):
    B, L = x.shape
    V, D = token_emb.shape
    NV = D // LANES
    CH = L // 2
    ROWS_W = B * L // NW
    NCH = ROWS_W // CH
    assert L == 200 and D == 128 and B % NW == 0 and NCH % RING == 0

    x3 = x.astype(jnp.int32).reshape(NW, NCH, CH)
    seg3 = seg.astype(jnp.int32).reshape(NW, ROWS_W)

    mesh = plsc.VectorSubcoreMesh(
        core_axis_name="c", subcore_axis_name="s",
        num_cores=NC, num_subcores=NS)

    @pl.kernel(
        out_type=jax.ShapeDtypeStruct((B * L // CH, CH, D), jnp.float32),
        mesh=mesh,
        scratch_types=[
            pltpu.VMEM((NCH, CH), jnp.int32),
            pltpu.VMEM((ROWS_W + LANES,), jnp.int32),
            pltpu.VMEM((2 * L, D), jnp.float32),
            pltpu.VMEM((CH, D), jnp.float32),
            pltpu.VMEM((CH, D), jnp.float32),
            pltpu.VMEM((CH, D), jnp.float32),
            pltpu.VMEM((CH, D), jnp.float32),
            pltpu.VMEM((D,), jnp.float32),
            pltpu.VMEM((D,), jnp.float32),
            pltpu.SemaphoreType.DMA,
            pltpu.SemaphoreType.DMA,
            pltpu.SemaphoreType.DMA,
            pltpu.SemaphoreType.DMA,
            pltpu.SemaphoreType.DMA,
            pltpu.SemaphoreType.DMA,
            pltpu.SemaphoreType.DMA,
            pltpu.SemaphoreType.DMA,
        ],
        compiler_params=pltpu.CompilerParams(needs_layout_passes=False),
    )
    def body(x_hbm, seg_hbm, tok_hbm, pos_hbm, gamma_hbm, beta_hbm, out_hbm,
             idx_all, seg_all, pc, b0, b1, b2, b3, gamma_v, beta_v,
             g0, g1, g2, g3, w0, w1, w2, w3):
        bufs = (b0, b1, b2, b3)
        gsem = (g0, g1, g2, g3)
        wsem = (w0, w1, w2, w3)
        wid = lax.axis_index("s") * NC + lax.axis_index("c")

        pltpu.sync_copy(x_hbm.at[wid], idx_all)
        pltpu.sync_copy(seg_hbm.at[wid], seg_all.at[pl.ds(0, ROWS_W)])
        pltpu.sync_copy(pos_hbm.at[pl.ds(0, L)], pc.at[pl.ds(0, L)])
        pltpu.sync_copy(gamma_hbm, gamma_v)
        pltpu.sync_copy(beta_hbm, beta_v)

        sls = [pl.ds(LANES * j, LANES) for j in range(NV)]
        p0 = [pc[0, sl] for sl in sls]
        p1 = [pc[1, sl] for sl in sls]
        gam = [gamma_v[sl] for sl in sls]
        bet = [beta_v[sl] for sl in sls]

        def fill(l, _):
            for j, sl in enumerate(sls):
                v = pc[l, sl]
                pc[L + l, sl] = v + p1[j]
                pc[l, sl] = v + p0[j]
            return 0
        lax.fori_loop(0, L, fill, 0)

        inv_d = jnp.float32(1.0 / D)
        eps = jnp.float32(1e-5)

        def gather_start(c, r):
            pltpu.async_copy(tok_hbm.at[idx_all.at[c]], bufs[r], gsem[r])

        def gather_wait(c, r):
            pltpu.make_async_copy(
                tok_hbm.at[idx_all.at[c]], bufs[r], gsem[r]).wait()

        def wb_wait(r):
            pltpu.make_async_copy(bufs[r], out_hbm.at[0], wsem[r]).wait()

        def compute_chunk(c, r):
            buf = bufs[r]
            cb = CH * c
            l0 = CH * (r % 2)

            def group(gidx, _):
                i0 = 4 * gidx
                sv = seg_all[pl.ds(cb + i0, LANES)]
                for k in range(4):
                    i = i0 + k
                    psel = sv[k] * L + (l0 + i)
                    h = [buf[i, sl] + pc[psel, sl] for sl in sls]
                    acc = (h[0] + h[1]) + (h[2] + h[3])
                    acc = acc + ((h[4] + h[5]) + (h[6] + h[7]))
                    accq = (h[0] * h[0] + h[1] * h[1]) + (h[2] * h[2] + h[3] * h[3])
                    accq = accq + ((h[4] * h[4] + h[5] * h[5])
                                   + (h[6] * h[6] + h[7] * h[7]))
                    mu = jnp.sum(acc) * inv_d
                    var = jnp.sum(accq) * inv_d - mu * mu
                    rs = _rsqrt_newton(var + eps)
                    mub = lax.broadcast_in_dim(mu, (LANES,), ())
                    rb = lax.broadcast_in_dim(rs, (LANES,), ())
                    for j, sl in enumerate(sls):
                        buf[i, sl] = (h[j] - mub) * rb * gam[j] + bet[j]
                return 0
            lax.fori_loop(0, CH // 4, group, 0)

        for r in range(RING - 1):
            gather_start(r, r)

        def t_body(t, _):
            for r in range(RING):
                c = RING * t + r
                gather_wait(c, r)
                compute_chunk(c, r)
                pltpu.async_copy(bufs[r], out_hbm.at[wid * NCH + c], wsem[r])
                cn = c + RING - 1
                rn = (r + RING - 1) % RING

                @pl.when(cn < NCH)
                def _():
                    @pl.when(cn >= RING)
                    def _():
                        wb_wait(rn)
                    gather_start(cn, rn)
            return 0
        lax.fori_loop(0, NCH // RING, t_body, 0)

        for r in range(RING):
            wb_wait(r)

    out = body(x3, seg3, token_emb, pos_emb, gamma, beta)
    return out.reshape(B, L, D)


# speedup vs baseline: 4.8098x; 1.0701x over previous
import jax
import jax.numpy as jnp
from jax import lax
from jax.experimental import pallas as pl
from jax.experimental.pallas import tpu as pltpu
from jax.experimental.pallas import tpu_sc as plsc

NC = 2
NS = 16
NW = NC * NS
LANES = 16
RING = 4


def _rsqrt_newton(v):
    bits = lax.bitcast_convert_type(v, jnp.int32)
    seed = jnp.int32(0x5F3759DF) - lax.shift_right_logical(bits, 1)
    y = lax.bitcast_convert_type(seed, jnp.float32)
    hv = jnp.float32(0.5) * v
    for _ in range(3):
        y = y * (jnp.float32(1.5) - hv * y * y)
    return y


def kernel(x, seg, token_emb, pos_emb, gamma, beta):
    B, L = x.shape
    V, D = token_emb.shape
    NV = D // LANES
    CH = L // 2
    ROWS_W = B * L // NW
    NCH = ROWS_W // CH
    assert L == 200 and D == 128 and B % NW == 0 and NCH % RING == 0

    x3 = x.astype(jnp.int32).reshape(NW, NCH, CH)
    seg3 = seg.astype(jnp.int32).reshape(NW, ROWS_W)

    mesh = plsc.VectorSubcoreMesh(
        core_axis_name="c", subcore_axis_name="s",
        num_cores=NC, num_subcores=NS)

    @pl.kernel(
        out_type=jax.ShapeDtypeStruct((B * L // CH, CH, D), jnp.float32),
        mesh=mesh,
        scratch_types=[
            pltpu.VMEM((NCH, CH), jnp.int32),
            pltpu.VMEM((ROWS_W + LANES,), jnp.int32),
            pltpu.VMEM((2 * L, D), jnp.float32),
            pltpu.VMEM((CH, D), jnp.float32),
            pltpu.VMEM((CH, D), jnp.float32),
            pltpu.VMEM((CH, D), jnp.float32),
            pltpu.VMEM((CH, D), jnp.float32),
            pltpu.VMEM((D,), jnp.float32),
            pltpu.VMEM((D,), jnp.float32),
            pltpu.VMEM((8 * LANES,), jnp.float32),
            pltpu.SemaphoreType.DMA,
            pltpu.SemaphoreType.DMA,
            pltpu.SemaphoreType.DMA,
            pltpu.SemaphoreType.DMA,
            pltpu.SemaphoreType.DMA,
            pltpu.SemaphoreType.DMA,
            pltpu.SemaphoreType.DMA,
            pltpu.SemaphoreType.DMA,
        ],
        compiler_params=pltpu.CompilerParams(needs_layout_passes=False),
    )
    def body(x_hbm, seg_hbm, tok_hbm, pos_hbm, gamma_hbm, beta_hbm, out_hbm,
             idx_all, seg_all, pc, b0, b1, b2, b3, gamma_v, beta_v, tmp,
             g0, g1, g2, g3, w0, w1, w2, w3):
        bufs = (b0, b1, b2, b3)
        gsem = (g0, g1, g2, g3)
        wsem = (w0, w1, w2, w3)
        wid = lax.axis_index("s") * NC + lax.axis_index("c")

        pltpu.sync_copy(x_hbm.at[wid], idx_all)
        pltpu.sync_copy(seg_hbm.at[wid], seg_all.at[pl.ds(0, ROWS_W)])
        pltpu.sync_copy(pos_hbm.at[pl.ds(0, L)], pc.at[pl.ds(0, L)])
        pltpu.sync_copy(gamma_hbm, gamma_v)
        pltpu.sync_copy(beta_hbm, beta_v)

        sls = [pl.ds(LANES * j, LANES) for j in range(NV)]
        p0 = [pc[0, sl] for sl in sls]
        p1 = [pc[1, sl] for sl in sls]
        gam = [gamma_v[sl] for sl in sls]
        bet = [beta_v[sl] for sl in sls]

        def fill(l, _):
            for j, sl in enumerate(sls):
                v = pc[l, sl]
                pc[L + l, sl] = v + p1[j]
                pc[l, sl] = v + p0[j]
            return 0
        lax.fori_loop(0, L, fill, 0)

        inv_d = jnp.float32(1.0 / D)
        eps = jnp.float32(1e-5)

        def gather_start(c, r):
            pltpu.async_copy(tok_hbm.at[idx_all.at[c]], bufs[r], gsem[r])

        def gather_wait(c, r):
            pltpu.make_async_copy(
                tok_hbm.at[idx_all.at[c]], bufs[r], gsem[r]).wait()

        def wb_wait(r):
            pltpu.make_async_copy(bufs[r], out_hbm.at[0], wsem[r]).wait()

        def compute_chunk(c, r):
            buf = bufs[r]
            cb = CH * c
            l0 = CH * (r % 2)

            def group(gidx, _):
                i0 = 4 * gidx
                sv = seg_all[pl.ds(cb + i0, LANES)]
                for k in range(4):
                    i = i0 + k
                    psel = sv[k] * L + (l0 + i)
                    h = [buf[i, sl] + pc[psel, sl] for sl in sls]
                    acc = (h[0] + h[1]) + (h[2] + h[3])
                    acc = acc + ((h[4] + h[5]) + (h[6] + h[7]))
                    accq = (h[0] * h[0] + h[1] * h[1]) + (h[2] * h[2] + h[3] * h[3])
                    accq = accq + ((h[4] * h[4] + h[5] * h[5])
                                   + (h[6] * h[6] + h[7] * h[7]))
                    tmp[pl.ds(32 * k, LANES)] = plsc.cumsum(acc)
                    tmp[pl.ds(32 * k + LANES, LANES)] = plsc.cumsum(accq)
                    mub = plsc.load_gather(
                        tmp, [jnp.full((LANES,), 32 * k + 15, jnp.int32)]) * inv_d
                    sqb = plsc.load_gather(
                        tmp, [jnp.full((LANES,), 32 * k + 31, jnp.int32)]) * inv_d
                    varb = sqb - mub * mub
                    rb = _rsqrt_newton(varb + eps)
                    for j, sl in enumerate(sls):
                        buf[i, sl] = (h[j] - mub) * rb * gam[j] + bet[j]
                return 0
            lax.fori_loop(0, CH // 4, group, 0)

        for r in range(RING - 1):
            gather_start(r, r)

        def t_body(t, _):
            for r in range(RING):
                c = RING * t + r
                gather_wait(c, r)
                compute_chunk(c, r)
                pltpu.async_copy(bufs[r], out_hbm.at[wid * NCH + c], wsem[r])
                cn = c + RING - 1
                rn = (r + RING - 1) % RING

                @pl.when(cn < NCH)
                def _():
                    @pl.when(cn >= RING)
                    def _():
                        wb_wait(rn)
                    gather_start(cn, rn)
            return 0
        lax.fori_loop(0, NCH // RING, t_body, 0)

        for r in range(RING):
            wb_wait(r)

    out = body(x3, seg3, token_emb, pos_emb, gamma, beta)
    return out.reshape(B, L, D)


# speedup vs baseline: 7.8103x; 1.6238x over previous
import jax
import jax.numpy as jnp
from jax import lax
from jax.experimental import pallas as pl
from jax.experimental.pallas import tpu as pltpu
from jax.experimental.pallas import tpu_sc as plsc

NC = 2
NS = 16
NW = NC * NS
LANES = 16
RING = 4


def _rsqrt_newton(v):
    bits = lax.bitcast_convert_type(v, jnp.int32)
    seed = jnp.int32(0x5F3759DF) - lax.shift_right_logical(bits, 1)
    y = lax.bitcast_convert_type(seed, jnp.float32)
    hv = jnp.float32(0.5) * v
    for _ in range(3):
        y = y * (jnp.float32(1.5) - hv * y * y)
    return y


def kernel(x, seg, token_emb, pos_emb, gamma, beta):
    B, L = x.shape
    V, D = token_emb.shape
    NV = D // LANES
    CH = L // 2
    ROWS_W = B * L // NW
    NCH = ROWS_W // CH
    assert L == 200 and D == 128 and B % NW == 0 and NCH % RING == 0

    x3 = x.astype(jnp.int32).reshape(NW, NCH, CH)
    seg3 = seg.astype(jnp.int32).reshape(NW, ROWS_W)

    mesh = plsc.VectorSubcoreMesh(
        core_axis_name="c", subcore_axis_name="s",
        num_cores=NC, num_subcores=NS)

    @pl.kernel(
        out_type=jax.ShapeDtypeStruct((B * L // CH, CH, D), jnp.float32),
        mesh=mesh,
        scratch_types=[
            pltpu.VMEM((NCH, CH), jnp.int32),
            pltpu.VMEM((ROWS_W + LANES,), jnp.int32),
            pltpu.VMEM((2 * L, D), jnp.float32),
            pltpu.VMEM((CH, D), jnp.float32),
            pltpu.VMEM((CH, D), jnp.float32),
            pltpu.VMEM((CH, D), jnp.float32),
            pltpu.VMEM((CH, D), jnp.float32),
            pltpu.VMEM((D,), jnp.float32),
            pltpu.VMEM((D,), jnp.float32),
            pltpu.SemaphoreType.DMA,
            pltpu.SemaphoreType.DMA,
            pltpu.SemaphoreType.DMA,
            pltpu.SemaphoreType.DMA,
            pltpu.SemaphoreType.DMA,
            pltpu.SemaphoreType.DMA,
            pltpu.SemaphoreType.DMA,
            pltpu.SemaphoreType.DMA,
        ],
        compiler_params=pltpu.CompilerParams(needs_layout_passes=False),
    )
    def body(x_hbm, seg_hbm, tok_hbm, pos_hbm, gamma_hbm, beta_hbm, out_hbm,
             idx_all, seg_all, pc, b0, b1, b2, b3, gamma_v, beta_v,
             g0, g1, g2, g3, w0, w1, w2, w3):
        bufs = (b0, b1, b2, b3)
        gsem = (g0, g1, g2, g3)
        wsem = (w0, w1, w2, w3)
        wid = lax.axis_index("s") * NC + lax.axis_index("c")

        pltpu.sync_copy(x_hbm.at[wid], idx_all)
        pltpu.sync_copy(seg_hbm.at[wid], seg_all.at[pl.ds(0, ROWS_W)])
        pltpu.sync_copy(pos_hbm.at[pl.ds(0, L)], pc.at[pl.ds(0, L)])
        pltpu.sync_copy(gamma_hbm, gamma_v)
        pltpu.sync_copy(beta_hbm, beta_v)

        sls = [pl.ds(LANES * j, LANES) for j in range(NV)]
        p0 = [pc[0, sl] for sl in sls]
        p1 = [pc[1, sl] for sl in sls]
        gam = [gamma_v[sl] for sl in sls]
        bet = [beta_v[sl] for sl in sls]

        def fill(l, _):
            for j, sl in enumerate(sls):
                v = pc[l, sl]
                pc[L + l, sl] = v + p1[j]
                pc[l, sl] = v + p0[j]
            return 0
        lax.fori_loop(0, L, fill, 0)

        inv_d = jnp.float32(1.0 / D)
        eps = jnp.float32(1e-5)

        def gather_start(c, r):
            pltpu.async_copy(tok_hbm.at[idx_all.at[c]], bufs[r], gsem[r])

        def gather_wait(c, r):
            pltpu.make_async_copy(
                tok_hbm.at[idx_all.at[c]], bufs[r], gsem[r]).wait()

        def wb_wait(r):
            pltpu.make_async_copy(bufs[r], out_hbm.at[0], wsem[r]).wait()

        def compute_chunk(c, r):
            buf = bufs[r]
            cb = CH * c
            l0 = CH * (r % 2)

            @plsc.parallel_loop(0, CH, unroll=4)
            def row(i):
                s = seg_all[pl.ds(cb + i, LANES)][0]
                psel = s * L + (l0 + i)
                h = [buf[i, sl] + pc[psel, sl] for sl in sls]
                acc = (h[0] + h[1]) + (h[2] + h[3])
                acc = acc + ((h[4] + h[5]) + (h[6] + h[7]))
                accq = (h[0] * h[0] + h[1] * h[1]) + (h[2] * h[2] + h[3] * h[3])
                accq = accq + ((h[4] * h[4] + h[5] * h[5])
                               + (h[6] * h[6] + h[7] * h[7]))
                mu = jnp.sum(acc) * inv_d
                var = jnp.sum(accq) * inv_d - mu * mu
                rs = _rsqrt_newton(var + eps)
                mub = lax.broadcast_in_dim(mu, (LANES,), ())
                rb = lax.broadcast_in_dim(rs, (LANES,), ())
                for j, sl in enumerate(sls):
                    buf[i, sl] = (h[j] - mub) * rb * gam[j] + bet[j]

        for r in range(RING - 1):
            gather_start(r, r)

        def t_body(t, _):
            for r in range(RING):
                c = RING * t + r
                gather_wait(c, r)
                compute_chunk(c, r)
                pltpu.async_copy(bufs[r], out_hbm.at[wid * NCH + c], wsem[r])
                cn = c + RING - 1
                rn = (r + RING - 1) % RING

                @pl.when(cn < NCH)
                def _():
                    @pl.when(cn >= RING)
                    def _():
                        wb_wait(rn)
                    gather_start(cn, rn)
            return 0
        lax.fori_loop(0, NCH // RING, t_body, 0)

        for r in range(RING):
            wb_wait(r)

    out = body(x3, seg3, token_emb, pos_emb, gamma, beta)
    return out.reshape(B, L, D)


# speedup vs baseline: 11.5269x; 1.4759x over previous
import jax
import jax.numpy as jnp
from jax import lax
from jax.experimental import pallas as pl
from jax.experimental.pallas import tpu as pltpu
from jax.experimental.pallas import tpu_sc as plsc

NC = 2
NS = 16
NW = NC * NS
LANES = 16
RING = 4
CHA = 104


def _rsqrt_newton(v):
    bits = lax.bitcast_convert_type(v, jnp.int32)
    seed = jnp.int32(0x5F3759DF) - lax.shift_right_logical(bits, 1)
    y = lax.bitcast_convert_type(seed, jnp.float32)
    hv = jnp.float32(0.5) * v
    for _ in range(3):
        y = y * (jnp.float32(1.5) - hv * y * y)
    return y


def kernel(x, seg, token_emb, pos_emb, gamma, beta):
    B, L = x.shape
    V, D = token_emb.shape
    NV = D // LANES
    CHB = L - CHA
    SEQ_W = B // NW
    ROWS_W = SEQ_W * L
    NCH = 2 * SEQ_W
    assert L == 200 and D == 128 and B % NW == 0 and NCH % RING == 0

    xi = x.astype(jnp.int32)
    xa = xi[:, :CHA].reshape(NW, SEQ_W, CHA)
    xb = xi[:, CHA:].reshape(NW, SEQ_W, CHB)
    seg3 = seg.astype(jnp.int32).reshape(NW, ROWS_W)

    mesh = plsc.VectorSubcoreMesh(
        core_axis_name="c", subcore_axis_name="s",
        num_cores=NC, num_subcores=NS)

    @pl.kernel(
        out_type=jax.ShapeDtypeStruct((B, L, D), jnp.float32),
        mesh=mesh,
        scratch_types=[
            pltpu.VMEM((SEQ_W, CHA), jnp.int32),
            pltpu.VMEM((SEQ_W, CHB), jnp.int32),
            pltpu.VMEM((ROWS_W + LANES,), jnp.int32),
            pltpu.VMEM((2 * L, D), jnp.float32),
            pltpu.VMEM((CHA, D), jnp.float32),
            pltpu.VMEM((CHA, D), jnp.float32),
            pltpu.VMEM((CHA, D), jnp.float32),
            pltpu.VMEM((CHA, D), jnp.float32),
            pltpu.VMEM((D,), jnp.float32),
            pltpu.VMEM((D,), jnp.float32),
            pltpu.SemaphoreType.DMA,
            pltpu.SemaphoreType.DMA,
            pltpu.SemaphoreType.DMA,
            pltpu.SemaphoreType.DMA,
            pltpu.SemaphoreType.DMA,
            pltpu.SemaphoreType.DMA,
            pltpu.SemaphoreType.DMA,
            pltpu.SemaphoreType.DMA,
        ],
        compiler_params=pltpu.CompilerParams(needs_layout_passes=False),
    )
    def body(xa_hbm, xb_hbm, seg_hbm, tok_hbm, pos_hbm, gamma_hbm, beta_hbm,
             out_hbm,
             idx_a, idx_b, seg_all, pc, b0, b1, b2, b3, gamma_v, beta_v,
             g0, g1, g2, g3, w0, w1, w2, w3):
        bufs = (b0, b1, b2, b3)
        gsem = (g0, g1, g2, g3)
        wsem = (w0, w1, w2, w3)
        wid = lax.axis_index("s") * NC + lax.axis_index("c")

        pltpu.sync_copy(xa_hbm.at[wid], idx_a)
        pltpu.sync_copy(xb_hbm.at[wid], idx_b)
        pltpu.sync_copy(seg_hbm.at[wid], seg_all.at[pl.ds(0, ROWS_W)])
        pltpu.sync_copy(pos_hbm.at[pl.ds(0, L)], pc.at[pl.ds(0, L)])
        pltpu.sync_copy(gamma_hbm, gamma_v)
        pltpu.sync_copy(beta_hbm, beta_v)

        sls = [pl.ds(LANES * j, LANES) for j in range(NV)]
        p0 = [pc[0, sl] for sl in sls]
        p1 = [pc[1, sl] for sl in sls]
        gam = [gamma_v[sl] for sl in sls]
        bet = [beta_v[sl] for sl in sls]

        def fill(l, _):
            for j, sl in enumerate(sls):
                v = pc[l, sl]
                pc[L + l, sl] = v + p1[j]
                pc[l, sl] = v + p0[j]
            return 0
        lax.fori_loop(0, L, fill, 0)

        inv_d = jnp.float32(1.0 / D)
        eps = jnp.float32(1e-5)

        def chrows(r):
            return CHA if r % 2 == 0 else CHB

        def bufsl(r):
            return bufs[r] if r % 2 == 0 else bufs[r].at[pl.ds(0, CHB)]

        def seq_of(c, r):
            del r
            return c // 2

        def gather_start(c, r):
            iab = idx_a if r % 2 == 0 else idx_b
            pltpu.async_copy(tok_hbm.at[iab.at[seq_of(c, r)]], bufsl(r), gsem[r])

        def gather_wait(c, r):
            iab = idx_a if r % 2 == 0 else idx_b
            pltpu.make_async_copy(
                tok_hbm.at[iab.at[seq_of(c, r)]], bufsl(r), gsem[r]).wait()

        def wb_wait(r):
            pltpu.make_async_copy(
                bufsl(r), out_hbm.at[0, pl.ds(0, chrows(r))], wsem[r]).wait()

        def compute_chunk(c, r):
            buf = bufs[r]
            l0 = CHA * (r % 2)
            cb = L * seq_of(c, r) + l0

            @plsc.parallel_loop(0, chrows(r), unroll=4)
            def row(i):
                s = seg_all[pl.ds(cb + i, LANES)][0]
                psel = s * L + (l0 + i)
                h = [buf[i, sl] + pc[psel, sl] for sl in sls]
                acc = (h[0] + h[1]) + (h[2] + h[3])
                acc = acc + ((h[4] + h[5]) + (h[6] + h[7]))
                accq = (h[0] * h[0] + h[1] * h[1]) + (h[2] * h[2] + h[3] * h[3])
                accq = accq + ((h[4] * h[4] + h[5] * h[5])
                               + (h[6] * h[6] + h[7] * h[7]))
                mu = jnp.sum(acc) * inv_d
                var = jnp.sum(accq) * inv_d - mu * mu
                rs = _rsqrt_newton(var + eps)
                mub = lax.broadcast_in_dim(mu, (LANES,), ())
                rb = lax.broadcast_in_dim(rs, (LANES,), ())
                for j, sl in enumerate(sls):
                    buf[i, sl] = (h[j] - mub) * rb * gam[j] + bet[j]

        for r in range(RING - 1):
            gather_start(r, r)

        def t_body(t, _):
            for r in range(RING):
                c = RING * t + r
                gather_wait(c, r)
                compute_chunk(c, r)
                gq = wid * SEQ_W + seq_of(c, r)
                pltpu.async_copy(
                    bufsl(r), out_hbm.at[gq, pl.ds(CHA * (r % 2), chrows(r))],
                    wsem[r])
                cn = c + RING - 1
                rn = (r + RING - 1) % RING

                @pl.when(cn < NCH)
                def _():
                    @pl.when(cn >= RING)
                    def _():
                        wb_wait(rn)
                    gather_start(cn, rn)
            return 0
        lax.fori_loop(0, NCH // RING, t_body, 0)

        for r in range(RING):
            wb_wait(r)

    return body(xa, xb, seg3, token_emb, pos_emb, gamma, beta)


# speedup vs baseline: 12.8614x; 1.1158x over previous
import jax
import jax.numpy as jnp
from jax import lax
from jax.experimental import pallas as pl
from jax.experimental.pallas import tpu as pltpu
from jax.experimental.pallas import tpu_sc as plsc

NC = 2
NS = 16
NW = NC * NS
LANES = 16
RING = 4
CHA = 104


def _rsqrt_newton(v):
    bits = lax.bitcast_convert_type(v, jnp.int32)
    seed = jnp.int32(0x5F3759DF) - lax.shift_right_logical(bits, 1)
    y = lax.bitcast_convert_type(seed, jnp.float32)
    hv = jnp.float32(0.5) * v
    for _ in range(3):
        y = y * (jnp.float32(1.5) - hv * y * y)
    return y


def kernel(x, seg, token_emb, pos_emb, gamma, beta):
    B, L = x.shape
    V, D = token_emb.shape
    NV = D // LANES
    CHB = L - CHA
    SEQ_W = B // NW
    ROWS_W = SEQ_W * L
    NCH = 2 * SEQ_W
    assert L == 200 and D == 128 and B % NW == 0 and NCH % RING == 0

    xi = x.astype(jnp.int32)
    xa = xi[:, :CHA].reshape(NW, SEQ_W, CHA)
    xb = xi[:, CHA:].reshape(NW, SEQ_W, CHB)
    seg3 = seg.astype(jnp.int32).reshape(NW, ROWS_W)

    mesh = plsc.VectorSubcoreMesh(
        core_axis_name="c", subcore_axis_name="s",
        num_cores=NC, num_subcores=NS)

    @pl.kernel(
        out_type=jax.ShapeDtypeStruct((B, L, D), jnp.float32),
        mesh=mesh,
        scratch_types=[
            pltpu.VMEM((SEQ_W, CHA), jnp.int32),
            pltpu.VMEM((SEQ_W, CHB), jnp.int32),
            pltpu.VMEM((ROWS_W + LANES,), jnp.int32),
            pltpu.VMEM((2 * L, D), jnp.float32),
            pltpu.VMEM((CHA, D), jnp.float32),
            pltpu.VMEM((CHA, D), jnp.float32),
            pltpu.VMEM((CHA, D), jnp.float32),
            pltpu.VMEM((CHA, D), jnp.float32),
            pltpu.VMEM((D,), jnp.float32),
            pltpu.VMEM((D,), jnp.float32),
            pltpu.SemaphoreType.DMA,
            pltpu.SemaphoreType.DMA,
            pltpu.SemaphoreType.DMA,
            pltpu.SemaphoreType.DMA,
            pltpu.SemaphoreType.DMA,
            pltpu.SemaphoreType.DMA,
            pltpu.SemaphoreType.DMA,
            pltpu.SemaphoreType.DMA,
        ],
        compiler_params=pltpu.CompilerParams(needs_layout_passes=False),
    )
    def body(xa_hbm, xb_hbm, seg_hbm, tok_hbm, pos_hbm, gamma_hbm, beta_hbm,
             out_hbm,
             idx_a, idx_b, seg_all, pc, b0, b1, b2, b3, gamma_v, beta_v,
             g0, g1, g2, g3, w0, w1, w2, w3):
        bufs = (b0, b1, b2, b3)
        gsem = (g0, g1, g2, g3)
        wsem = (w0, w1, w2, w3)
        wid = lax.axis_index("s") * NC + lax.axis_index("c")

        pltpu.sync_copy(xa_hbm.at[wid], idx_a)
        pltpu.sync_copy(xb_hbm.at[wid], idx_b)
        pltpu.sync_copy(seg_hbm.at[wid], seg_all.at[pl.ds(0, ROWS_W)])
        pltpu.sync_copy(pos_hbm.at[pl.ds(0, L)], pc.at[pl.ds(0, L)])
        pltpu.sync_copy(gamma_hbm, gamma_v)
        pltpu.sync_copy(beta_hbm, beta_v)

        sls = [pl.ds(LANES * j, LANES) for j in range(NV)]
        p0 = [pc[0, sl] for sl in sls]
        p1 = [pc[1, sl] for sl in sls]
        gam = [gamma_v[sl] for sl in sls]
        bet = [beta_v[sl] for sl in sls]

        def fill(l, _):
            for j, sl in enumerate(sls):
                v = pc[l, sl]
                pc[L + l, sl] = v + p1[j]
                pc[l, sl] = v + p0[j]
            return 0
        lax.fori_loop(0, L, fill, 0)

        inv_d = jnp.float32(1.0 / D)
        eps = jnp.float32(1e-5)

        def chrows(r):
            return CHA if r % 2 == 0 else CHB

        def bufsl(r):
            return bufs[r] if r % 2 == 0 else bufs[r].at[pl.ds(0, CHB)]

        def seq_of(c, r):
            del r
            return c // 2

        def gather_start(c, r):
            iab = idx_a if r % 2 == 0 else idx_b
            pltpu.async_copy(tok_hbm.at[iab.at[seq_of(c, r)]], bufsl(r), gsem[r])

        def gather_wait(c, r):
            iab = idx_a if r % 2 == 0 else idx_b
            pltpu.make_async_copy(
                tok_hbm.at[iab.at[seq_of(c, r)]], bufsl(r), gsem[r]).wait()

        def wb_wait(r):
            pltpu.make_async_copy(
                bufsl(r), out_hbm.at[0, pl.ds(0, chrows(r))], wsem[r]).wait()

        one = jnp.float32(1.0)
        zero = jnp.float32(0.0)
        nontriv = plsc.all_reduce_population_count(gam[0] != one)
        for j in range(NV):
            if j:
                nontriv = nontriv + plsc.all_reduce_population_count(
                    gam[j] != one)
            nontriv = nontriv + plsc.all_reduce_population_count(
                bet[j] != zero)
        nontriv_s = nontriv[0]

        def compute_chunk(c, r):
            buf = bufs[r]
            l0 = CHA * (r % 2)
            cb = L * seq_of(c, r) + l0

            def make_row(affine):
                def row(i):
                    s = seg_all[pl.ds(cb + i, LANES)][0]
                    psel = s * L + (l0 + i)
                    h = [buf[i, sl] + pc[psel, sl] for sl in sls]
                    acc = (h[0] + h[1]) + (h[2] + h[3])
                    acc = acc + ((h[4] + h[5]) + (h[6] + h[7]))
                    accq = (h[0] * h[0] + h[1] * h[1]) + (h[2] * h[2] + h[3] * h[3])
                    accq = accq + ((h[4] * h[4] + h[5] * h[5])
                                   + (h[6] * h[6] + h[7] * h[7]))
                    mu = jnp.sum(acc) * inv_d
                    var = jnp.sum(accq) * inv_d - mu * mu
                    rs = _rsqrt_newton(var + eps)
                    mub = lax.broadcast_in_dim(mu, (LANES,), ())
                    rb = lax.broadcast_in_dim(rs, (LANES,), ())
                    for j, sl in enumerate(sls):
                        y = (h[j] - mub) * rb
                        buf[i, sl] = y * gam[j] + bet[j] if affine else y
                return row

            @pl.when(nontriv_s == 0)
            def _():
                plsc.parallel_loop(0, chrows(r), unroll=4)(make_row(False))

            @pl.when(nontriv_s != 0)
            def _():
                plsc.parallel_loop(0, chrows(r), unroll=2)(make_row(True))

        for r in range(RING - 1):
            gather_start(r, r)

        def t_body(t, _):
            for r in range(RING):
                c = RING * t + r
                gather_wait(c, r)
                compute_chunk(c, r)
                gq = wid * SEQ_W + seq_of(c, r)
                pltpu.async_copy(
                    bufsl(r), out_hbm.at[gq, pl.ds(CHA * (r % 2), chrows(r))],
                    wsem[r])
                cn = c + RING - 1
                rn = (r + RING - 1) % RING

                @pl.when(cn < NCH)
                def _():
                    @pl.when(cn >= RING)
                    def _():
                        wb_wait(rn)
                    gather_start(cn, rn)
            return 0
        lax.fori_loop(0, NCH // RING, t_body, 0)

        for r in range(RING):
            wb_wait(r)

    return body(xa, xb, seg3, token_emb, pos_emb, gamma, beta)


# speedup vs baseline: 13.3127x; 1.0351x over previous
import jax
import jax.numpy as jnp
from jax import lax
from jax.experimental import pallas as pl
from jax.experimental.pallas import tpu as pltpu
from jax.experimental.pallas import tpu_sc as plsc

NC = 2
NS = 16
NW = NC * NS
LANES = 16
RING = 4
CHA = 104


def _rsqrt_newton(v):
    bits = lax.bitcast_convert_type(v, jnp.int32)
    seed = jnp.int32(0x5F3759DF) - lax.shift_right_logical(bits, 1)
    y = lax.bitcast_convert_type(seed, jnp.float32)
    hv = jnp.float32(0.5) * v
    for _ in range(3):
        y = y * (jnp.float32(1.5) - hv * y * y)
    return y


def kernel(x, seg, token_emb, pos_emb, gamma, beta):
    B, L = x.shape
    V, D = token_emb.shape
    NV = D // LANES
    CHB = L - CHA
    SEQ_W = B // NW
    ROWS_W = SEQ_W * L
    NCH = 2 * SEQ_W
    assert L == 200 and D == 128 and B % NW == 0 and NCH % RING == 0

    xi = x.astype(jnp.int32)
    xa = xi[:, :CHA].reshape(NW, SEQ_W, CHA)
    xb = xi[:, CHA:].reshape(NW, SEQ_W, CHB)
    seg3 = seg.astype(jnp.int32).reshape(NW, ROWS_W)

    mesh = plsc.VectorSubcoreMesh(
        core_axis_name="c", subcore_axis_name="s",
        num_cores=NC, num_subcores=NS)

    @pl.kernel(
        out_type=jax.ShapeDtypeStruct((B, L, D), jnp.float32),
        mesh=mesh,
        scratch_types=[
            pltpu.VMEM((SEQ_W, CHA), jnp.int32),
            pltpu.VMEM((SEQ_W, CHB), jnp.int32),
            pltpu.VMEM((ROWS_W + LANES,), jnp.int32),
            pltpu.VMEM((2 * L, D), jnp.float32),
            pltpu.VMEM((CHA, D), jnp.float32),
            pltpu.VMEM((CHA, D), jnp.float32),
            pltpu.VMEM((CHA, D), jnp.float32),
            pltpu.VMEM((CHA, D), jnp.float32),
            pltpu.VMEM((D,), jnp.float32),
            pltpu.VMEM((D,), jnp.float32),
            pltpu.SemaphoreType.DMA,
            pltpu.SemaphoreType.DMA,
            pltpu.SemaphoreType.DMA,
            pltpu.SemaphoreType.DMA,
            pltpu.SemaphoreType.DMA,
            pltpu.SemaphoreType.DMA,
            pltpu.SemaphoreType.DMA,
            pltpu.SemaphoreType.DMA,
        ],
        compiler_params=pltpu.CompilerParams(needs_layout_passes=False),
    )
    def body(xa_hbm, xb_hbm, seg_hbm, tok_hbm, pos_hbm, gamma_hbm, beta_hbm,
             out_hbm,
             idx_a, idx_b, seg_all, pc, b0, b1, b2, b3, gamma_v, beta_v,
             g0, g1, g2, g3, w0, w1, w2, w3):
        bufs = (b0, b1, b2, b3)
        gsem = (g0, g1, g2, g3)
        wsem = (w0, w1, w2, w3)
        wid = lax.axis_index("s") * NC + lax.axis_index("c")

        pltpu.sync_copy(xa_hbm.at[wid], idx_a)
        pltpu.sync_copy(xb_hbm.at[wid], idx_b)
        pltpu.sync_copy(seg_hbm.at[wid], seg_all.at[pl.ds(0, ROWS_W)])
        pltpu.sync_copy(pos_hbm.at[pl.ds(0, L)], pc.at[pl.ds(0, L)])
        pltpu.sync_copy(gamma_hbm, gamma_v)
        pltpu.sync_copy(beta_hbm, beta_v)

        sls = [pl.ds(LANES * j, LANES) for j in range(NV)]
        p0 = [pc[0, sl] for sl in sls]
        p1 = [pc[1, sl] for sl in sls]
        gam = [gamma_v[sl] for sl in sls]
        bet = [beta_v[sl] for sl in sls]

        def fill(l, _):
            for j, sl in enumerate(sls):
                v = pc[l, sl]
                pc[L + l, sl] = v + p1[j]
                pc[l, sl] = v + p0[j]
            return 0
        lax.fori_loop(0, L, fill, 0)

        inv_d = jnp.float32(1.0 / D)
        eps = jnp.float32(1e-5)

        def chrows(r):
            return CHA if r % 2 == 0 else CHB

        def bufsl(r):
            return bufs[r] if r % 2 == 0 else bufs[r].at[pl.ds(0, CHB)]

        def seq_of(c, r):
            del r
            return c // 2

        def gather_start(c, r):
            iab = idx_a if r % 2 == 0 else idx_b
            pltpu.async_copy(tok_hbm.at[iab.at[seq_of(c, r)]], bufsl(r), gsem[r])

        def gather_wait(c, r):
            iab = idx_a if r % 2 == 0 else idx_b
            pltpu.make_async_copy(
                tok_hbm.at[iab.at[seq_of(c, r)]], bufsl(r), gsem[r]).wait()

        def wb_wait(r):
            pltpu.make_async_copy(
                bufsl(r), out_hbm.at[0, pl.ds(0, chrows(r))], wsem[r]).wait()

        one = jnp.float32(1.0)
        zero = jnp.float32(0.0)
        nontriv = plsc.all_reduce_population_count(gam[0] != one)
        for j in range(NV):
            if j:
                nontriv = nontriv + plsc.all_reduce_population_count(
                    gam[j] != one)
            nontriv = nontriv + plsc.all_reduce_population_count(
                bet[j] != zero)
        nontriv_s = nontriv[0]

        def compute_chunk(c, r):
            buf = bufs[r]
            l0 = CHA * (r % 2)
            cb = L * seq_of(c, r) + l0

            def one_row(i, s, affine):
                psel = s * L + (l0 + i)
                h = [buf[i, sl] + pc[psel, sl] for sl in sls]
                acc = (h[0] + h[1]) + (h[2] + h[3])
                acc = acc + ((h[4] + h[5]) + (h[6] + h[7]))
                accq = (h[0] * h[0] + h[1] * h[1]) + (h[2] * h[2] + h[3] * h[3])
                accq = accq + ((h[4] * h[4] + h[5] * h[5])
                               + (h[6] * h[6] + h[7] * h[7]))
                mu = jnp.sum(acc) * inv_d
                var = jnp.sum(accq) * inv_d - mu * mu
                rs = _rsqrt_newton(var + eps)
                mub = lax.broadcast_in_dim(mu, (LANES,), ())
                rb = lax.broadcast_in_dim(rs, (LANES,), ())
                for j, sl in enumerate(sls):
                    y = (h[j] - mub) * rb
                    buf[i, sl] = y * gam[j] + bet[j] if affine else y

            @pl.when(nontriv_s == 0)
            def _():
                @plsc.parallel_loop(0, chrows(r), step=4)
                def row4(i0):
                    sv = seg_all[pl.ds(cb + i0, LANES)]
                    for k in range(4):
                        one_row(i0 + k, sv[k], False)

            @pl.when(nontriv_s != 0)
            def _():
                @plsc.parallel_loop(0, chrows(r), unroll=2)
                def row1(i):
                    one_row(i, seg_all[pl.ds(cb + i, LANES)][0], True)

        for r in range(RING - 1):
            gather_start(r, r)

        def t_body(t, _):
            for r in range(RING):
                c = RING * t + r
                gather_wait(c, r)
                compute_chunk(c, r)
                gq = wid * SEQ_W + seq_of(c, r)
                pltpu.async_copy(
                    bufsl(r), out_hbm.at[gq, pl.ds(CHA * (r % 2), chrows(r))],
                    wsem[r])
                cn = c + RING - 1
                rn = (r + RING - 1) % RING

                @pl.when(cn < NCH)
                def _():
                    @pl.when(cn >= RING)
                    def _():
                        wb_wait(rn)
                    gather_start(cn, rn)
            return 0
        lax.fori_loop(0, NCH // RING, t_body, 0)

        for r in range(RING):
            wb_wait(r)

    return body(xa, xb, seg3, token_emb, pos_emb, gamma, beta)


# speedup vs baseline: 13.3377x; 1.0019x over previous
import jax
import jax.numpy as jnp
from jax import lax
from jax.experimental import pallas as pl
from jax.experimental.pallas import tpu as pltpu
from jax.experimental.pallas import tpu_sc as plsc

NC = 2
NS = 16
NW = NC * NS
LANES = 16
RING = 4
CHA = 104


def _rsqrt_newton(v):
    bits = lax.bitcast_convert_type(v, jnp.int32)
    seed = jnp.int32(0x5F3759DF) - lax.shift_right_logical(bits, 1)
    y = lax.bitcast_convert_type(seed, jnp.float32)
    hv = jnp.float32(0.5) * v
    for _ in range(3):
        y = y * (jnp.float32(1.5) - hv * y * y)
    return y


def kernel(x, seg, token_emb, pos_emb, gamma, beta):
    B, L = x.shape
    V, D = token_emb.shape
    NV = D // LANES
    CHB = L - CHA
    SEQ_W = B // NW
    ROWS_W = SEQ_W * L
    NCH = 2 * SEQ_W
    assert L == 200 and D == 128 and B % NW == 0 and NCH % RING == 0

    xi = x.astype(jnp.int32)
    xa = xi[:, :CHA].reshape(NW, SEQ_W, CHA)
    xb = xi[:, CHA:].reshape(NW, SEQ_W, CHB)
    seg3 = seg.astype(jnp.int32).reshape(NW, ROWS_W)

    mesh = plsc.VectorSubcoreMesh(
        core_axis_name="c", subcore_axis_name="s",
        num_cores=NC, num_subcores=NS)

    @pl.kernel(
        out_type=jax.ShapeDtypeStruct((B, L, D), jnp.float32),
        mesh=mesh,
        scratch_types=[
            pltpu.VMEM((SEQ_W, CHA), jnp.int32),
            pltpu.VMEM((SEQ_W, CHB), jnp.int32),
            pltpu.VMEM((ROWS_W + LANES,), jnp.int32),
            pltpu.VMEM((2 * L, D), jnp.float32),
            pltpu.VMEM((CHA, D), jnp.float32),
            pltpu.VMEM((CHA, D), jnp.float32),
            pltpu.VMEM((CHA, D), jnp.float32),
            pltpu.VMEM((CHA, D), jnp.float32),
            pltpu.VMEM((D,), jnp.float32),
            pltpu.VMEM((D,), jnp.float32),
            pltpu.SemaphoreType.DMA,
            pltpu.SemaphoreType.DMA,
            pltpu.SemaphoreType.DMA,
            pltpu.SemaphoreType.DMA,
            pltpu.SemaphoreType.DMA,
            pltpu.SemaphoreType.DMA,
            pltpu.SemaphoreType.DMA,
            pltpu.SemaphoreType.DMA,
        ],
        compiler_params=pltpu.CompilerParams(needs_layout_passes=False),
    )
    def body(xa_hbm, xb_hbm, seg_hbm, tok_hbm, pos_hbm, gamma_hbm, beta_hbm,
             out_hbm,
             idx_a, idx_b, seg_all, pc, b0, b1, b2, b3, gamma_v, beta_v,
             g0, g1, g2, g3, w0, w1, w2, w3):
        bufs = (b0, b1, b2, b3)
        gsem = (g0, g1, g2, g3)
        wsem = (w0, w1, w2, w3)
        wid = lax.axis_index("s") * NC + lax.axis_index("c")

        pltpu.sync_copy(xa_hbm.at[wid], idx_a)
        pltpu.sync_copy(xb_hbm.at[wid], idx_b)

        sls = [pl.ds(LANES * j, LANES) for j in range(NV)]
        def gather_prime(c, r):
            iab = idx_a if r % 2 == 0 else idx_b
            dst = bufs[r] if r % 2 == 0 else bufs[r].at[pl.ds(0, CHB)]
            pltpu.async_copy(tok_hbm.at[iab.at[c // 2]], dst, gsem[r])

        for r in range(RING - 1):
            gather_prime(r, r)

        pltpu.sync_copy(seg_hbm.at[wid], seg_all.at[pl.ds(0, ROWS_W)])
        pltpu.sync_copy(pos_hbm.at[pl.ds(0, L)], pc.at[pl.ds(0, L)])
        pltpu.sync_copy(gamma_hbm, gamma_v)
        pltpu.sync_copy(beta_hbm, beta_v)

        p0 = [pc[0, sl] for sl in sls]
        p1 = [pc[1, sl] for sl in sls]
        gam = [gamma_v[sl] for sl in sls]
        bet = [beta_v[sl] for sl in sls]

        @plsc.parallel_loop(0, L, unroll=2)
        def fill(l):
            for j, sl in enumerate(sls):
                v = pc[l, sl]
                pc[L + l, sl] = v + p1[j]
                pc[l, sl] = v + p0[j]

        inv_d = jnp.float32(1.0 / D)
        eps = jnp.float32(1e-5)

        def chrows(r):
            return CHA if r % 2 == 0 else CHB

        def bufsl(r):
            return bufs[r] if r % 2 == 0 else bufs[r].at[pl.ds(0, CHB)]

        def seq_of(c, r):
            del r
            return c // 2

        def gather_start(c, r):
            iab = idx_a if r % 2 == 0 else idx_b
            pltpu.async_copy(tok_hbm.at[iab.at[seq_of(c, r)]], bufsl(r), gsem[r])

        def gather_wait(c, r):
            iab = idx_a if r % 2 == 0 else idx_b
            pltpu.make_async_copy(
                tok_hbm.at[iab.at[seq_of(c, r)]], bufsl(r), gsem[r]).wait()

        def wb_wait(r):
            pltpu.make_async_copy(
                bufsl(r), out_hbm.at[0, pl.ds(0, chrows(r))], wsem[r]).wait()

        one = jnp.float32(1.0)
        zero = jnp.float32(0.0)
        nontriv = plsc.all_reduce_population_count(gam[0] != one)
        for j in range(NV):
            if j:
                nontriv = nontriv + plsc.all_reduce_population_count(
                    gam[j] != one)
            nontriv = nontriv + plsc.all_reduce_population_count(
                bet[j] != zero)
        nontriv_s = nontriv[0]

        def compute_chunk(c, r):
            buf = bufs[r]
            l0 = CHA * (r % 2)
            cb = L * seq_of(c, r) + l0

            def one_row(i, s, affine):
                psel = s * L + (l0 + i)
                h = [buf[i, sl] + pc[psel, sl] for sl in sls]
                acc = (h[0] + h[1]) + (h[2] + h[3])
                acc = acc + ((h[4] + h[5]) + (h[6] + h[7]))
                accq = (h[0] * h[0] + h[1] * h[1]) + (h[2] * h[2] + h[3] * h[3])
                accq = accq + ((h[4] * h[4] + h[5] * h[5])
                               + (h[6] * h[6] + h[7] * h[7]))
                mu = jnp.sum(acc) * inv_d
                var = jnp.sum(accq) * inv_d - mu * mu
                rs = _rsqrt_newton(var + eps)
                mub = lax.broadcast_in_dim(mu, (LANES,), ())
                rb = lax.broadcast_in_dim(rs, (LANES,), ())
                for j, sl in enumerate(sls):
                    y = (h[j] - mub) * rb
                    buf[i, sl] = y * gam[j] + bet[j] if affine else y

            @pl.when(nontriv_s == 0)
            def _():
                @plsc.parallel_loop(0, chrows(r), step=4)
                def row4(i0):
                    sv = seg_all[pl.ds(cb + i0, LANES)]
                    for k in range(4):
                        one_row(i0 + k, sv[k], False)

            @pl.when(nontriv_s != 0)
            def _():
                @plsc.parallel_loop(0, chrows(r), unroll=2)
                def row1(i):
                    one_row(i, seg_all[pl.ds(cb + i, LANES)][0], True)

        def t_body(t, _):
            for r in range(RING):
                c = RING * t + r
                gather_wait(c, r)
                compute_chunk(c, r)
                gq = wid * SEQ_W + seq_of(c, r)
                pltpu.async_copy(
                    bufsl(r), out_hbm.at[gq, pl.ds(CHA * (r % 2), chrows(r))],
                    wsem[r])
                cn = c + RING - 1
                rn = (r + RING - 1) % RING

                @pl.when(cn < NCH)
                def _():
                    @pl.when(cn >= RING)
                    def _():
                        wb_wait(rn)
                    gather_start(cn, rn)
            return 0
        lax.fori_loop(0, NCH // RING, t_body, 0)

        for r in range(RING):
            wb_wait(r)

    return body(xa, xb, seg3, token_emb, pos_emb, gamma, beta)
